# Initial kernel scaffold; baseline (speedup 1.0000x reference)
import jax
import jax.numpy as jnp
from jax.experimental import pallas as pl


def kernel(x, adj_indices, adj_values, W):
    raise NotImplementedError("write your pallas kernel here")


# speedup vs baseline: 2.0966x; 2.0966x over previous
import functools

import jax
import jax.numpy as jnp
from jax import lax
from jax.experimental import pallas as pl
from jax.experimental.pallas import tpu as pltpu
from jax.experimental.pallas import tpu_sc as plsc


def _mm_body(x_ref, w0_ref, w1_ref, o0_ref, o1_ref):
    o0_ref[...] = jnp.dot(x_ref[...], w0_ref[...],
                          preferred_element_type=jnp.float32)
    o1_ref[...] = jnp.dot(x_ref[...], w1_ref[...],
                          preferred_element_type=jnp.float32)


def _matmul_split(x, W):
    n, d_in = x.shape
    d_out = W.shape[1]
    dh = d_out // 2
    blk = 2000
    grid = (n // blk,)
    return pl.pallas_call(
        _mm_body,
        grid=grid,
        in_specs=[
            pl.BlockSpec((blk, d_in), lambda i: (i, 0)),
            pl.BlockSpec((d_in, dh), lambda i: (0, 0)),
            pl.BlockSpec((d_in, dh), lambda i: (0, 0)),
        ],
        out_specs=[
            pl.BlockSpec((blk, dh), lambda i: (i, 0)),
            pl.BlockSpec((blk, dh), lambda i: (i, 0)),
        ],
        out_shape=[
            jax.ShapeDtypeStruct((n, dh), jnp.float32),
            jax.ShapeDtypeStruct((n, dh), jnp.float32),
        ],
    )(x, W[:, :dh], W[:, dh:])


def _comb_body(p0_ref, p1_ref, o_ref):
    dh = p0_ref.shape[-1]
    o_ref[:, :dh] = jnp.maximum(p0_ref[...], 0.0)
    o_ref[:, dh:] = jnp.maximum(p1_ref[...], 0.0)


def _combine_relu(p0, p1):
    n, dh = p0.shape
    blk = 2000
    grid = (n // blk,)
    return pl.pallas_call(
        _comb_body,
        grid=grid,
        in_specs=[
            pl.BlockSpec((blk, dh), lambda i: (i, 0)),
            pl.BlockSpec((blk, dh), lambda i: (i, 0)),
        ],
        out_specs=pl.BlockSpec((blk, 2 * dh), lambda i: (i, 0)),
        out_shape=jax.ShapeDtypeStruct((n, 2 * dh), jnp.float32),
    )(p0, p1)


def _make_sc_scatter(n, e, dh, nc, ns):
    eps = e // ns
    chunk = 80
    nchunk = eps // chunk
    stripe = (n // ns) // 8 * 8
    tail_base = stripe * ns
    tail = n - tail_base
    assert eps * ns == e and nchunk * chunk == eps
    assert tail % 8 == 0 and stripe % 8 == 0
    dslices = dh // 16

    mesh = plsc.VectorSubcoreMesh(core_axis_name="c", subcore_axis_name="s")

    @functools.partial(
        pl.kernel,
        out_type=(
            jax.ShapeDtypeStruct((n, dh), jnp.float32),
            jax.ShapeDtypeStruct((n, dh), jnp.float32),
        ),
        mesh=mesh,
        compiler_params=pltpu.CompilerParams(use_tc_tiling_on_sc=False),
        scratch_types=[
            pltpu.VMEM_SHARED((n, dh), jnp.float32),
            pltpu.VMEM((stripe, dh), jnp.float32),
            pltpu.VMEM((chunk,), jnp.int32),
            pltpu.VMEM((chunk,), jnp.int32),
            pltpu.VMEM((chunk,), jnp.float32),
            pltpu.VMEM((chunk, dh), jnp.float32),
            pltpu.SemaphoreType.DMA,
        ],
    )
    def sc_scatter(h0_hbm, h1_hbm, src_hbm, dst_hbm, val_hbm,
                   out0_hbm, out1_hbm,
                   acc, zbuf, src_v, dst_v, val_v, rows_v, sem):
        cid = lax.axis_index("c")
        sid = lax.axis_index("s")

        def zero_body(i, _):
            for ds_i in range(dslices):
                zbuf[i, pl.ds(ds_i * 16, 16)] = jnp.zeros((16,), jnp.float32)
            return 0
        lax.fori_loop(0, stripe, zero_body, 0)
        base_row = pl.multiple_of(sid * stripe, 8)
        pltpu.sync_copy(zbuf, acc.at[pl.ds(base_row, stripe)])

        @pl.when(sid == 0)
        def _zero_tail():
            pltpu.sync_copy(zbuf.at[pl.ds(0, tail)],
                            acc.at[pl.ds(tail_base, tail)])
        plsc.subcore_barrier()

        ebase = sid * eps

        def chunk_body(k, _):
            off = pl.multiple_of(ebase + k * chunk, 8)
            pltpu.sync_copy(src_hbm.at[pl.ds(off, chunk)], src_v)

            @pl.when(cid == 0)
            def _g0():
                pltpu.async_copy(h0_hbm.at[src_v], rows_v, sem).wait()

            @pl.when(cid == 1)
            def _g1():
                pltpu.async_copy(h1_hbm.at[src_v], rows_v, sem).wait()

            pltpu.sync_copy(val_hbm.at[pl.ds(off, chunk)], val_v)
            pltpu.sync_copy(dst_hbm.at[pl.ds(off, chunk)], dst_v)

            def group_body(g, _):
                v_grp = val_v[pl.ds(g * 16, 16)]
                for j in range(16):
                    sp = jnp.broadcast_to(v_grp[j], (16,))
                    ei = g * 16 + j
                    for ds_i in range(dslices):
                        seg = rows_v[ei, pl.ds(ds_i * 16, 16)]
                        rows_v[ei, pl.ds(ds_i * 16, 16)] = seg * sp
                return 0
            lax.fori_loop(0, chunk // 16, group_body, 0)

            pltpu.sync_copy(rows_v, acc.at[dst_v], add=True)
            return 0
        lax.fori_loop(0, nchunk, chunk_body, 0)

        plsc.subcore_barrier()

        @pl.when(cid == 0)
        def _d0():
            pltpu.sync_copy(acc.at[pl.ds(base_row, stripe)],
                            out0_hbm.at[pl.ds(base_row, stripe)])

            @pl.when(sid == 0)
            def _d0t():
                pltpu.sync_copy(acc.at[pl.ds(tail_base, tail)],
                                out0_hbm.at[pl.ds(tail_base, tail)])

        @pl.when(cid == 1)
        def _d1():
            pltpu.sync_copy(acc.at[pl.ds(base_row, stripe)],
                            out1_hbm.at[pl.ds(base_row, stripe)])

            @pl.when(sid == 0)
            def _d1t():
                pltpu.sync_copy(acc.at[pl.ds(tail_base, tail)],
                                out1_hbm.at[pl.ds(tail_base, tail)])

    return sc_scatter


def kernel(x, adj_indices, adj_values, W):
    n, d_in = x.shape
    d_out = W.shape[1]
    e = adj_values.shape[0]

    info = plsc.get_sparse_core_info()
    nc, ns = info.num_cores, info.num_subcores

    h0, h1 = _matmul_split(x, W)
    dst = adj_indices[0]
    src = adj_indices[1]
    sc = _make_sc_scatter(n, e, d_out // 2, nc, ns)
    p0, p1 = sc(h0, h1, src, dst, adj_values)
    return _combine_relu(p0, p1)


# speedup vs baseline: 8.4757x; 4.0426x over previous
import functools

import jax
import jax.numpy as jnp
from jax import lax
from jax.experimental import pallas as pl
from jax.experimental.pallas import tpu as pltpu
from jax.experimental.pallas import tpu_sc as plsc


def _mm_body(x_ref, w0_ref, w1_ref, o0_ref, o1_ref):
    o0_ref[...] = jnp.dot(x_ref[...], w0_ref[...],
                          preferred_element_type=jnp.float32)
    o1_ref[...] = jnp.dot(x_ref[...], w1_ref[...],
                          preferred_element_type=jnp.float32)


def _matmul_split(x, W):
    n, d_in = x.shape
    d_out = W.shape[1]
    dh = d_out // 2
    blk = 2000
    grid = (n // blk,)
    return pl.pallas_call(
        _mm_body,
        grid=grid,
        in_specs=[
            pl.BlockSpec((blk, d_in), lambda i: (i, 0)),
            pl.BlockSpec((d_in, dh), lambda i: (0, 0)),
            pl.BlockSpec((d_in, dh), lambda i: (0, 0)),
        ],
        out_specs=[
            pl.BlockSpec((blk, dh), lambda i: (i, 0)),
            pl.BlockSpec((blk, dh), lambda i: (i, 0)),
        ],
        out_shape=[
            jax.ShapeDtypeStruct((n, dh), jnp.float32),
            jax.ShapeDtypeStruct((n, dh), jnp.float32),
        ],
    )(x, W[:, :dh], W[:, dh:])


def _comb_body(p0_ref, p1_ref, o_ref):
    dh = p0_ref.shape[-1]
    o_ref[:, :dh] = jnp.maximum(p0_ref[...], 0.0)
    o_ref[:, dh:] = jnp.maximum(p1_ref[...], 0.0)


def _combine_relu(p0, p1):
    n, dh = p0.shape
    blk = 2000
    grid = (n // blk,)
    return pl.pallas_call(
        _comb_body,
        grid=grid,
        in_specs=[
            pl.BlockSpec((blk, dh), lambda i: (i, 0)),
            pl.BlockSpec((blk, dh), lambda i: (i, 0)),
        ],
        out_specs=pl.BlockSpec((blk, 2 * dh), lambda i: (i, 0)),
        out_shape=jax.ShapeDtypeStruct((n, 2 * dh), jnp.float32),
    )(p0, p1)


def _make_sc_scatter(n, e, dh, nc, ns):
    eps = e // ns
    chunk = 80
    nchunk = eps // chunk
    stripe = (n // ns) // 8 * 8
    tail_base = stripe * ns
    tail = n - tail_base
    zrows = 208
    assert eps * ns == e and nchunk * chunk == eps
    assert tail % 8 == 0 and stripe % zrows == 0 and tail <= zrows
    dslices = dh // 16

    mesh = plsc.VectorSubcoreMesh(core_axis_name="c", subcore_axis_name="s")

    @functools.partial(
        pl.kernel,
        out_type=(
            jax.ShapeDtypeStruct((n, dh), jnp.float32),
            jax.ShapeDtypeStruct((n, dh), jnp.float32),
        ),
        mesh=mesh,
        compiler_params=pltpu.CompilerParams(use_tc_tiling_on_sc=False),
        scratch_types=[
            pltpu.VMEM_SHARED((n, dh), jnp.float32),
            pltpu.VMEM((zrows, dh), jnp.float32),
            pltpu.VMEM((nchunk, chunk), jnp.int32),
            pltpu.VMEM((nchunk, chunk), jnp.int32),
            pltpu.VMEM((nchunk, chunk), jnp.float32),
            pltpu.VMEM((chunk, dh), jnp.float32),
            pltpu.VMEM((chunk, dh), jnp.float32),
            pltpu.SemaphoreType.DMA,
            pltpu.SemaphoreType.DMA,
        ],
    )
    def sc_scatter(h0_hbm, h1_hbm, src_hbm, dst_hbm, val_hbm,
                   out0_hbm, out1_hbm,
                   acc, zbuf, src_v, dst_v, val_v, rows0, rows1, sem0, sem1):
        cid = lax.axis_index("c")
        sid = lax.axis_index("s")
        rows = (rows0, rows1)
        sems = (sem0, sem1)

        pltpu.sync_copy(src_hbm.at[sid], src_v)
        pltpu.sync_copy(dst_hbm.at[sid], dst_v)
        pltpu.sync_copy(val_hbm.at[sid], val_v)

        def zero_body(i, _):
            for ds_i in range(dslices):
                zbuf[i, pl.ds(ds_i * 16, 16)] = jnp.zeros((16,), jnp.float32)
            return 0
        lax.fori_loop(0, zrows, zero_body, 0)
        base_row = pl.multiple_of(sid * stripe, 8)
        for z in range(stripe // zrows):
            pltpu.sync_copy(zbuf, acc.at[pl.ds(base_row + z * zrows, zrows)])

        @pl.when(sid == 0)
        def _zero_tail():
            pltpu.sync_copy(zbuf.at[pl.ds(0, tail)],
                            acc.at[pl.ds(tail_base, tail)])
        plsc.subcore_barrier()

        h_hbm = (h0_hbm, h1_hbm)

        def issue_gather(k, buf):
            for c in range(nc):
                @pl.when(cid == c)
                def _g():
                    pltpu.async_copy(h_hbm[c].at[src_v.at[k]],
                                     rows[buf], sems[buf])

        def wait_gather(k, buf):
            for c in range(nc):
                @pl.when(cid == c)
                def _w():
                    pltpu.make_async_copy(h_hbm[c].at[src_v.at[k]],
                                          rows[buf], sems[buf]).wait()

        issue_gather(0, 0)

        def chunk_body(k, _):
            buf = lax.rem(k, 2)

            @pl.when(k + 1 < nchunk)
            def _prefetch():
                for b in range(2):
                    @pl.when(buf == b)
                    def _i():
                        issue_gather(k + 1, 1 - b)

            for b in range(2):
                @pl.when(buf == b)
                def _run():
                    wait_gather(k, b)
                    rbuf = rows[b]
                    for g in range(chunk // 16):
                        v_grp = val_v[k, pl.ds(g * 16, 16)]
                        for j in range(16):
                            sp = jnp.broadcast_to(v_grp[j], (16,))
                            ei = g * 16 + j
                            for ds_i in range(dslices):
                                seg = rbuf[ei, pl.ds(ds_i * 16, 16)]
                                rbuf[ei, pl.ds(ds_i * 16, 16)] = seg * sp
                    pltpu.sync_copy(rbuf, acc.at[dst_v.at[k]], add=True)
            return 0
        lax.fori_loop(0, nchunk, chunk_body, 0)

        plsc.subcore_barrier()

        @pl.when(cid == 0)
        def _d0():
            pltpu.sync_copy(acc.at[pl.ds(base_row, stripe)],
                            out0_hbm.at[pl.ds(base_row, stripe)])

            @pl.when(sid == 0)
            def _d0t():
                pltpu.sync_copy(acc.at[pl.ds(tail_base, tail)],
                                out0_hbm.at[pl.ds(tail_base, tail)])

        @pl.when(cid == 1)
        def _d1():
            pltpu.sync_copy(acc.at[pl.ds(base_row, stripe)],
                            out1_hbm.at[pl.ds(base_row, stripe)])

            @pl.when(sid == 0)
            def _d1t():
                pltpu.sync_copy(acc.at[pl.ds(tail_base, tail)],
                                out1_hbm.at[pl.ds(tail_base, tail)])

    return sc_scatter


def kernel(x, adj_indices, adj_values, W):
    n, d_in = x.shape
    d_out = W.shape[1]
    e = adj_values.shape[0]

    info = plsc.get_sparse_core_info()
    nc, ns = info.num_cores, info.num_subcores

    h0, h1 = _matmul_split(x, W)
    eps = e // ns
    chunk = 80
    nchunk = eps // chunk
    dst = adj_indices[0].reshape(ns, nchunk, chunk)
    src = adj_indices[1].reshape(ns, nchunk, chunk)
    vals = adj_values.reshape(ns, nchunk, chunk)
    sc = _make_sc_scatter(n, e, d_out // 2, nc, ns)
    p0, p1 = sc(h0, h1, src, dst, vals)
    return _combine_relu(p0, p1)
